# Initial kernel scaffold; baseline (speedup 1.0000x reference)
import jax
import jax.numpy as jnp
from jax import lax
from jax.experimental import pallas as pl
from jax.experimental.pallas import tpu as pltpu
from jax.experimental.pallas import tpu_sc as plsc

N_BATCH = 16
N_ATOMS = 512
N_NEIGH = 48
N_RADIUS = 16
N_CHANNEL = 16
CUTOFF = 6.0
N_Z = 10
L = 16
NW = 32
APW = N_BATCH * N_ATOMS // NW
RC = N_RADIUS * N_CHANNEL

_GDN = lax.GatherDimensionNumbers(
    offset_dims=(), collapsed_slice_dims=(0,), start_index_map=(0,))


def _bcast(vec, idx_vec):
    return lax.gather(vec, idx_vec[:, None], _GDN, (1,),
                      mode=lax.GatherScatterMode.PROMISE_IN_BOUNDS)


def _cos_poly(u):
    c = jnp.float32(-1.0 / 87178291200.0)
    c = c * u + jnp.float32(1.0 / 479001600.0)
    c = c * u + jnp.float32(-1.0 / 3628800.0)
    c = c * u + jnp.float32(1.0 / 40320.0)
    c = c * u + jnp.float32(-1.0 / 720.0)
    c = c * u + jnp.float32(1.0 / 24.0)
    c = c * u + jnp.float32(-0.5)
    return c * u + jnp.float32(1.0)


def _sc_body(coord_hbm, anum_hbm, nbr_hbm, emb_hbm, negeta_hbm, rss_hbm,
             out_hbm, cxv, cyv, czv, anv, nbv, rssv, negv, gv, ov, embs):
    s = lax.axis_index("s")
    c = lax.axis_index("c")
    wid = s * 2 + c
    b = wid // 2
    h = wid % 2

    pltpu.sync_copy(coord_hbm.at[b, 0], cxv)
    pltpu.sync_copy(coord_hbm.at[b, 1], cyv)
    pltpu.sync_copy(coord_hbm.at[b, 2], czv)
    pltpu.sync_copy(anum_hbm.at[b], anv)
    nbase = (b * N_ATOMS + h * APW) * N_NEIGH
    pltpu.sync_copy(nbr_hbm.at[pl.ds(nbase, APW * N_NEIGH)], nbv)
    pltpu.sync_copy(emb_hbm, embs)
    pltpu.sync_copy(rss_hbm, rssv)
    pltpu.sync_copy(negeta_hbm, negv)

    rss_vec = rssv[...]
    neg_vec = negv[...]
    iota16 = lax.iota(jnp.int32, L)
    iota_rc = iota16 * N_CHANNEL
    ks = [jnp.full((L,), k, jnp.int32) for k in range(L)]
    zero16 = jnp.zeros((L,), jnp.float32)
    half = jnp.float32(0.5)
    three_half = jnp.float32(1.5)
    magic = jnp.int32(0x5F3759DF)

    def body(i, carry):
        nb0 = i * N_NEIGH
        for z in range(N_Z):
            gv[pl.ds(z * L, L)] = zero16
        n_i = h * APW + i
        own = jnp.full((L,), n_i, jnp.int32)
        xi = plsc.load_gather(cxv, [own])
        yi = plsc.load_gather(cyv, [own])
        zi = plsc.load_gather(czv, [own])
        for g in range(N_NEIGH // L):
            nbr = nbv[pl.ds(nb0 + g * L, L)]
            zng = plsc.load_gather(anv, [nbr])
            xj = plsc.load_gather(cxv, [nbr])
            yj = plsc.load_gather(cyv, [nbr])
            zj = plsc.load_gather(czv, [nbr])
            dx = xj - xi
            dy = yj - yi
            dz = zj - zi
            d2 = dx * dx + dy * dy + dz * dz + jnp.float32(1e-12)
            y = plsc.bitcast(magic - (plsc.bitcast(d2, jnp.int32) >> 1),
                             jnp.float32)
            hd2 = half * d2
            y = y * (three_half - hd2 * y * y)
            y = y * (three_half - hd2 * y * y)
            y = y * (three_half - hd2 * y * y)
            dd = d2 * y
            inb = d2 < jnp.float32(CUTOFF * CUTOFF)
            dcl = jnp.minimum(dd, jnp.float32(CUTOFF))
            x = dcl * jnp.float32(3.141592653589793 / CUTOFF)
            cosv = _cos_poly(x * x)
            cut = jnp.where(inb, half * (cosv + jnp.float32(1.0)), zero16)
            for k in range(L):
                db = _bcast(dcl, ks[k])
                cb = _bcast(cut, ks[k])
                zb = _bcast(zng, ks[k])
                tt = db - rss_vec
                fk = jnp.exp(tt * tt * neg_vec) * cb
                plsc.addupdate_scatter(gv, [zb * L + iota16], fk)
        gz = [gv[pl.ds(z * L, L)] for z in range(N_Z)]
        ob = i * RC
        for ch in range(N_CHANNEL):
            acc = gz[0] * embs[0, ch]
            for z in range(1, N_Z):
                acc = acc + gz[z] * embs[z, ch]
            plsc.store_scatter(ov, [iota_rc + (ob + ch)], acc)
        return carry

    lax.fori_loop(0, APW, body, 0)
    obase = (b * N_ATOMS + h * APW) * RC
    pltpu.sync_copy(ov, out_hbm.at[pl.ds(obase, APW * RC)])


@jax.jit
def _run(coord_t, anum, nbr_flat, emb_table, negeta, rss):
    mesh = plsc.VectorSubcoreMesh(core_axis_name="c", subcore_axis_name="s")
    f = pl.kernel(
        _sc_body,
        out_type=jax.ShapeDtypeStruct((N_BATCH * N_ATOMS * RC,), jnp.float32),
        mesh=mesh,
        scratch_types=[
            pltpu.VMEM((N_ATOMS,), jnp.float32),
            pltpu.VMEM((N_ATOMS,), jnp.float32),
            pltpu.VMEM((N_ATOMS,), jnp.float32),
            pltpu.VMEM((N_ATOMS,), jnp.int32),
            pltpu.VMEM((APW * N_NEIGH,), jnp.int32),
            pltpu.VMEM((L,), jnp.float32),
            pltpu.VMEM((L,), jnp.float32),
            pltpu.VMEM((N_Z * L,), jnp.float32),
            pltpu.VMEM((APW * RC,), jnp.float32),
            pltpu.SMEM((N_Z, N_CHANNEL), jnp.float32),
        ],
    )
    return f(coord_t, anum, nbr_flat, emb_table, negeta, rss)


def kernel(coordinate, atomic_number, neighbor, emb_table, etas, rss):
    coord_t = coordinate.astype(jnp.float32).transpose(0, 2, 1)
    anum = atomic_number.astype(jnp.int32)
    nbr_flat = neighbor.astype(jnp.int32).reshape(-1)
    out = _run(coord_t, anum, nbr_flat, emb_table.astype(jnp.float32),
               (-etas).astype(jnp.float32), rss.astype(jnp.float32))
    return out.reshape(N_BATCH, N_ATOMS, RC)


# speedup vs baseline: 48.8965x; 48.8965x over previous
import jax
import jax.numpy as jnp
from jax import lax
from jax.experimental import pallas as pl
from jax.experimental.pallas import tpu as pltpu
from jax.experimental.pallas import tpu_sc as plsc

N_BATCH = 16
N_ATOMS = 512
N_NEIGH = 48
N_RADIUS = 16
N_CHANNEL = 16
CUTOFF = 6.0
N_Z = 10
L = 16
NW = 32
APW = N_BATCH * N_ATOMS // NW
RC = N_RADIUS * N_CHANNEL

_GDN = lax.GatherDimensionNumbers(
    offset_dims=(), collapsed_slice_dims=(0,), start_index_map=(0,))


def _bcast(vec, idx_vec):
    return lax.gather(vec, idx_vec[:, None], _GDN, (1,),
                      mode=lax.GatherScatterMode.PROMISE_IN_BOUNDS)


def _cos_poly(u):
    c = jnp.float32(-1.0 / 87178291200.0)
    c = c * u + jnp.float32(1.0 / 479001600.0)
    c = c * u + jnp.float32(-1.0 / 3628800.0)
    c = c * u + jnp.float32(1.0 / 40320.0)
    c = c * u + jnp.float32(-1.0 / 720.0)
    c = c * u + jnp.float32(1.0 / 24.0)
    c = c * u + jnp.float32(-0.5)
    return c * u + jnp.float32(1.0)


def _sc_body(coord_hbm, anum_hbm, nbr_hbm, emb_hbm, negeta_hbm, rss_hbm,
             out_hbm, cxv, cyv, czv, anv, nbv, rssv, negv, gv, ov, embs):
    s = lax.axis_index("s")
    c = lax.axis_index("c")
    wid = s * 2 + c
    b = wid // 2
    h = wid % 2

    cbase = b * 3 * N_ATOMS
    pltpu.sync_copy(coord_hbm.at[pl.ds(cbase, N_ATOMS)], cxv)
    pltpu.sync_copy(coord_hbm.at[pl.ds(cbase + N_ATOMS, N_ATOMS)], cyv)
    pltpu.sync_copy(coord_hbm.at[pl.ds(cbase + 2 * N_ATOMS, N_ATOMS)], czv)
    pltpu.sync_copy(anum_hbm.at[pl.ds(b * N_ATOMS, N_ATOMS)], anv)
    nbase = (b * N_ATOMS + h * APW) * N_NEIGH
    pltpu.sync_copy(nbr_hbm.at[pl.ds(nbase, APW * N_NEIGH)], nbv)
    pltpu.sync_copy(emb_hbm, embs)
    pltpu.sync_copy(rss_hbm, rssv)
    pltpu.sync_copy(negeta_hbm, negv)

    rss_vec = rssv[...]
    neg_vec = negv[...]
    iota16 = lax.iota(jnp.int32, L)
    iota_rc = iota16 * N_CHANNEL
    ks = [jnp.full((L,), k, jnp.int32) for k in range(L)]
    zero16 = jnp.zeros((L,), jnp.float32)
    half = jnp.float32(0.5)
    three_half = jnp.float32(1.5)
    magic = jnp.int32(0x5F3759DF)

    def body(i, carry):
        nb0 = i * N_NEIGH
        for z in range(N_Z):
            gv[pl.ds(z * L, L)] = zero16
        n_i = h * APW + i
        own = jnp.full((L,), n_i, jnp.int32)
        xi = plsc.load_gather(cxv, [own])
        yi = plsc.load_gather(cyv, [own])
        zi = plsc.load_gather(czv, [own])
        for g in range(N_NEIGH // L):
            nbr = nbv[pl.ds(nb0 + g * L, L)]
            zng = plsc.load_gather(anv, [nbr])
            xj = plsc.load_gather(cxv, [nbr])
            yj = plsc.load_gather(cyv, [nbr])
            zj = plsc.load_gather(czv, [nbr])
            dx = xj - xi
            dy = yj - yi
            dz = zj - zi
            d2 = dx * dx + dy * dy + dz * dz + jnp.float32(1e-12)
            y = plsc.bitcast(magic - (plsc.bitcast(d2, jnp.int32) >> 1),
                             jnp.float32)
            hd2 = half * d2
            y = y * (three_half - hd2 * y * y)
            y = y * (three_half - hd2 * y * y)
            y = y * (three_half - hd2 * y * y)
            dd = d2 * y
            inb = d2 < jnp.float32(CUTOFF * CUTOFF)
            dcl = jnp.minimum(dd, jnp.float32(CUTOFF))
            x = dcl * jnp.float32(3.141592653589793 / CUTOFF)
            cosv = _cos_poly(x * x)
            cut = jnp.where(inb, half * (cosv + jnp.float32(1.0)), zero16)
            for k in range(L):
                db = _bcast(dcl, ks[k])
                cb = _bcast(cut, ks[k])
                zb = _bcast(zng, ks[k])
                tt = db - rss_vec
                fk = jnp.exp(tt * tt * neg_vec) * cb
                plsc.addupdate_scatter(gv, [zb * L + iota16], fk)
        gz = [gv[pl.ds(z * L, L)] for z in range(N_Z)]
        ob = i * RC
        for ch in range(N_CHANNEL):
            acc = gz[0] * embs[pl.ds(ch * L, L)]
            for z in range(1, N_Z):
                acc = acc + gz[z] * embs[pl.ds((z * N_CHANNEL + ch) * L, L)]
            plsc.store_scatter(ov, [iota_rc + (ob + ch)], acc)
        return carry

    lax.fori_loop(0, APW, body, 0)
    obase = (b * N_ATOMS + h * APW) * RC
    pltpu.sync_copy(ov, out_hbm.at[pl.ds(obase, APW * RC)])


@jax.jit
def _run(coord_t, anum, nbr_flat, emb_table, negeta, rss):
    mesh = plsc.VectorSubcoreMesh(core_axis_name="c", subcore_axis_name="s")
    f = pl.kernel(
        _sc_body,
        out_type=jax.ShapeDtypeStruct((N_BATCH * N_ATOMS * RC,), jnp.float32),
        mesh=mesh,
        compiler_params=pltpu.CompilerParams(needs_layout_passes=False),
        scratch_types=[
            pltpu.VMEM((N_ATOMS,), jnp.float32),
            pltpu.VMEM((N_ATOMS,), jnp.float32),
            pltpu.VMEM((N_ATOMS,), jnp.float32),
            pltpu.VMEM((N_ATOMS,), jnp.int32),
            pltpu.VMEM((APW * N_NEIGH,), jnp.int32),
            pltpu.VMEM((L,), jnp.float32),
            pltpu.VMEM((L,), jnp.float32),
            pltpu.VMEM((N_Z * L,), jnp.float32),
            pltpu.VMEM((APW * RC,), jnp.float32),
            pltpu.VMEM((N_Z * N_CHANNEL * L,), jnp.float32),
        ],
    )
    return f(coord_t, anum, nbr_flat, emb_table, negeta, rss)


def kernel(coordinate, atomic_number, neighbor, emb_table, etas, rss):
    coord_t = coordinate.astype(jnp.float32).transpose(0, 2, 1).reshape(-1)
    anum = atomic_number.astype(jnp.int32).reshape(-1)
    nbr_flat = neighbor.astype(jnp.int32).reshape(-1)
    emb_splat = jnp.repeat(
        emb_table.astype(jnp.float32).reshape(-1)[:, None], L, axis=1
    ).reshape(-1)
    out = _run(coord_t, anum, nbr_flat, emb_splat,
               (-etas).astype(jnp.float32), rss.astype(jnp.float32))
    return out.reshape(N_BATCH, N_ATOMS, RC)


# speedup vs baseline: 51.6422x; 1.0562x over previous
import jax
import jax.numpy as jnp
from jax import lax
from jax.experimental import pallas as pl
from jax.experimental.pallas import tpu as pltpu
from jax.experimental.pallas import tpu_sc as plsc

N_BATCH = 16
N_ATOMS = 512
N_NEIGH = 48
N_RADIUS = 16
N_CHANNEL = 16
CUTOFF = 6.0
N_Z = 10
L = 16
NW = 32
APW = N_BATCH * N_ATOMS // NW
RC = N_RADIUS * N_CHANNEL

_GDN = lax.GatherDimensionNumbers(
    offset_dims=(), collapsed_slice_dims=(0,), start_index_map=(0,))


def _bcast(vec, idx_vec):
    return lax.gather(vec, idx_vec[:, None], _GDN, (1,),
                      mode=lax.GatherScatterMode.PROMISE_IN_BOUNDS)


def _cos_poly(u):
    c = jnp.float32(-1.0 / 87178291200.0)
    c = c * u + jnp.float32(1.0 / 479001600.0)
    c = c * u + jnp.float32(-1.0 / 3628800.0)
    c = c * u + jnp.float32(1.0 / 40320.0)
    c = c * u + jnp.float32(-1.0 / 720.0)
    c = c * u + jnp.float32(1.0 / 24.0)
    c = c * u + jnp.float32(-0.5)
    return c * u + jnp.float32(1.0)


def _sc_body(coord_hbm, anum_hbm, nbr_hbm, emb_hbm, negeta_hbm, rss_hbm,
             out_hbm, cxv, cyv, czv, anv, nbv, rssv, negv, gv, ov, embs):
    s = lax.axis_index("s")
    c = lax.axis_index("c")
    wid = s * 2 + c
    b = wid // 2
    h = wid % 2

    cbase = b * 3 * N_ATOMS
    pltpu.sync_copy(coord_hbm.at[pl.ds(cbase, N_ATOMS)], cxv)
    pltpu.sync_copy(coord_hbm.at[pl.ds(cbase + N_ATOMS, N_ATOMS)], cyv)
    pltpu.sync_copy(coord_hbm.at[pl.ds(cbase + 2 * N_ATOMS, N_ATOMS)], czv)
    pltpu.sync_copy(anum_hbm.at[pl.ds(b * N_ATOMS, N_ATOMS)], anv)
    nbase = (b * N_ATOMS + h * APW) * N_NEIGH
    pltpu.sync_copy(nbr_hbm.at[pl.ds(nbase, APW * N_NEIGH)], nbv)
    pltpu.sync_copy(emb_hbm, embs)
    pltpu.sync_copy(rss_hbm, rssv)
    pltpu.sync_copy(negeta_hbm, negv)

    rss_vec = rssv[...]
    neg_vec = negv[...]
    iota16 = lax.iota(jnp.int32, L)
    iota_rc = iota16 * N_CHANNEL
    ks = [jnp.full((L,), k, jnp.int32) for k in range(L)]
    zero16 = jnp.zeros((L,), jnp.float32)
    half = jnp.float32(0.5)
    three_half = jnp.float32(1.5)
    magic = jnp.int32(0x5F3759DF)

    def atom(i, gb):
        nb0 = i * N_NEIGH
        for z in range(N_Z):
            gv[pl.ds(gb + z * L, L)] = zero16
        n_i = h * APW + i
        own = jnp.full((L,), n_i, jnp.int32)
        xi = plsc.load_gather(cxv, [own])
        yi = plsc.load_gather(cyv, [own])
        zi = plsc.load_gather(czv, [own])
        for g in range(N_NEIGH // L):
            nbr = nbv[pl.ds(nb0 + g * L, L)]
            zng = plsc.load_gather(anv, [nbr])
            xj = plsc.load_gather(cxv, [nbr])
            yj = plsc.load_gather(cyv, [nbr])
            zj = plsc.load_gather(czv, [nbr])
            dx = xj - xi
            dy = yj - yi
            dz = zj - zi
            d2 = (dx * dx + dy * dy) + (dz * dz + jnp.float32(1e-12))
            y = plsc.bitcast(magic - (plsc.bitcast(d2, jnp.int32) >> 1),
                             jnp.float32)
            hd2 = half * d2
            y = y * (three_half - hd2 * y * y)
            y = y * (three_half - hd2 * y * y)
            y = y * (three_half - hd2 * y * y)
            dd = d2 * y
            inb = d2 < jnp.float32(CUTOFF * CUTOFF)
            dcl = jnp.minimum(dd, jnp.float32(CUTOFF))
            x = dcl * jnp.float32(3.141592653589793 / CUTOFF)
            cosv = _cos_poly(x * x)
            cut = jnp.where(inb, half * (cosv + jnp.float32(1.0)), zero16)
            zidx = zng if gb == 0 else zng + jnp.int32(gb)
            for k in range(L):
                db = _bcast(dcl, ks[k])
                cb = _bcast(cut, ks[k])
                zb = _bcast(zidx, ks[k])
                tt = db - rss_vec
                fk = jnp.exp(tt * tt * neg_vec) * cb
                plsc.addupdate_scatter(gv, [zb + iota16], fk)

    def matmul(i, gb):
        gz = [gv[pl.ds(gb + z * L, L)] for z in range(N_Z)]
        ob = i * RC
        for ch in range(N_CHANNEL):
            p = [gz[z] * embs[pl.ds((z * N_CHANNEL + ch) * L, L)]
                 for z in range(N_Z)]
            while len(p) > 1:
                p = [p[j] + p[j + 1] for j in range(0, len(p) - 1, 2)] \
                    + ([p[-1]] if len(p) % 2 else [])
            plsc.store_scatter(ov, [iota_rc + (ob + ch)], p[0])

    def body(i, carry):
        i0 = i * 2
        i1 = i0 + 1
        atom(i0, 0)
        atom(i1, N_Z * L)
        matmul(i0, 0)
        matmul(i1, N_Z * L)
        return carry

    lax.fori_loop(0, APW // 2, body, 0)
    obase = (b * N_ATOMS + h * APW) * RC
    pltpu.sync_copy(ov, out_hbm.at[pl.ds(obase, APW * RC)])


@jax.jit
def _run(coord_t, anum, nbr_flat, emb_table, negeta, rss):
    mesh = plsc.VectorSubcoreMesh(core_axis_name="c", subcore_axis_name="s")
    f = pl.kernel(
        _sc_body,
        out_type=jax.ShapeDtypeStruct((N_BATCH * N_ATOMS * RC,), jnp.float32),
        mesh=mesh,
        compiler_params=pltpu.CompilerParams(needs_layout_passes=False),
        scratch_types=[
            pltpu.VMEM((N_ATOMS,), jnp.float32),
            pltpu.VMEM((N_ATOMS,), jnp.float32),
            pltpu.VMEM((N_ATOMS,), jnp.float32),
            pltpu.VMEM((N_ATOMS,), jnp.int32),
            pltpu.VMEM((APW * N_NEIGH,), jnp.int32),
            pltpu.VMEM((L,), jnp.float32),
            pltpu.VMEM((L,), jnp.float32),
            pltpu.VMEM((2 * N_Z * L,), jnp.float32),
            pltpu.VMEM((APW * RC,), jnp.float32),
            pltpu.VMEM((N_Z * N_CHANNEL * L,), jnp.float32),
        ],
    )
    return f(coord_t, anum, nbr_flat, emb_table, negeta, rss)


def kernel(coordinate, atomic_number, neighbor, emb_table, etas, rss):
    coord_t = coordinate.astype(jnp.float32).transpose(0, 2, 1).reshape(-1)
    anum = (atomic_number.astype(jnp.int32) * L).reshape(-1)
    nbr_flat = neighbor.astype(jnp.int32).reshape(-1)
    emb_splat = jnp.repeat(
        emb_table.astype(jnp.float32).reshape(-1)[:, None], L, axis=1
    ).reshape(-1)
    out = _run(coord_t, anum, nbr_flat, emb_splat,
               (-etas).astype(jnp.float32), rss.astype(jnp.float32))
    return out.reshape(N_BATCH, N_ATOMS, RC)
